# baseline (speedup 1.0000x reference)
import functools

import jax
import jax.numpy as jnp
from jax import lax
from jax.experimental import pallas as pl
from jax.experimental.pallas import tpu as pltpu
from jax.experimental.pallas import tpu_sc as plsc

N = 10000
D_IN = 128
D_HID = 16
D_OUT = 128
E = 320000

NC = 2
NS = 16
LANES = 128
NP = 10240
EP = 327680
G = EP // (NC * NS * LANES)
G_SC0 = 116
G_SC1 = 2 * G - G_SC0
RPT = NP // NS

_MESH = plsc.VectorSubcoreMesh(
    core_axis_name="c", subcore_axis_name="s", num_cores=NC, num_subcores=NS)


def _stage_idx(ei_hbm, idx_v, c, s):
    @pl.when(c == 0)
    def _():
        pltpu.sync_copy(ei_hbm.at[pl.ds(s * G_SC0, G_SC0)], idx_v)

    @pl.when(c == 1)
    def _():
        pltpu.sync_copy(ei_hbm.at[pl.ds(NS * G_SC0 + s * G_SC1, G_SC1)],
                        idx_v.at[pl.ds(0, G_SC1)])


def _deg_body(ei_hbm, zeros_hbm, ones_hbm, out_hbm, dst_v, ones_v, acc_sh,
              dsem):
    c = lax.axis_index("c")
    s = lax.axis_index("s")
    my_g = lax.select(c == 0, G_SC0, G_SC1)
    _stage_idx(ei_hbm, dst_v, c, s)
    pltpu.sync_copy(zeros_hbm.at[pl.ds(s * RPT, RPT)],
                    acc_sh.at[pl.ds(s * RPT, RPT)])
    pltpu.sync_copy(ones_hbm, ones_v)
    plsc.subcore_barrier()

    def step(g, carry):
        pltpu.async_copy(ones_v, acc_sh.at[dst_v.at[g]], dsem, add=True)
        pltpu.make_async_copy(ones_v, acc_sh.at[dst_v.at[g]], dsem).wait()
        return carry

    lax.fori_loop(0, my_g, step, 0)
    plsc.subcore_barrier()
    pltpu.sync_copy(acc_sh.at[pl.ds(s * RPT, RPT)],
                    out_hbm.at[c, pl.ds(s * RPT, RPT)])


_deg_call = functools.partial(
    pl.kernel, _deg_body, mesh=_MESH,
    compiler_params=pltpu.CompilerParams(use_tc_tiling_on_sc=False),
    out_type=jax.ShapeDtypeStruct((NC, NP, D_HID), jnp.float32),
    scratch_types=[
        pltpu.VMEM((G_SC0, LANES), jnp.int32),
        pltpu.VMEM((LANES, D_HID), jnp.float32),
        pltpu.VMEM_SHARED((NP, D_HID), jnp.float32),
        pltpu.SemaphoreType.DMA,
    ])()


_NB = 4


def _agg_body(u_hbm, src_hbm, dst_hbm, zeros_hbm, out_hbm,
              src_v, dst_v, rows_v, acc_sh, gsem, ssem):
    c = lax.axis_index("c")
    s = lax.axis_index("s")
    my_g = lax.select(c == 0, G_SC0, G_SC1)
    _stage_idx(src_hbm, src_v, c, s)
    _stage_idx(dst_hbm, dst_v, c, s)
    pltpu.sync_copy(zeros_hbm.at[pl.ds(s * RPT, RPT)],
                    acc_sh.at[pl.ds(s * RPT, RPT)])
    plsc.subcore_barrier()

    def step(g, carry):
        pltpu.async_copy(u_hbm.at[src_v.at[g]], rows_v.at[0], gsem.at[0])
        pltpu.make_async_copy(
            u_hbm.at[src_v.at[g]], rows_v.at[0], gsem.at[0]).wait()
        pltpu.async_copy(
            rows_v.at[0], acc_sh.at[dst_v.at[g]], ssem.at[0], add=True)
        pltpu.make_async_copy(
            rows_v.at[0], acc_sh.at[dst_v.at[g]], ssem.at[0]).wait()
        return carry

    lax.fori_loop(0, my_g, step, 0)
    plsc.subcore_barrier()
    pltpu.sync_copy(acc_sh.at[pl.ds(s * RPT, RPT)],
                    out_hbm.at[c, pl.ds(s * RPT, RPT)])


_agg_call = functools.partial(
    pl.kernel, _agg_body, mesh=_MESH,
    compiler_params=pltpu.CompilerParams(use_tc_tiling_on_sc=False),
    out_type=jax.ShapeDtypeStruct((NC, NP, D_HID), jnp.float32),
    scratch_types=[
        pltpu.VMEM((G_SC0, LANES), jnp.int32),
        pltpu.VMEM((G_SC0, LANES), jnp.int32),
        pltpu.VMEM((_NB, LANES, D_HID), jnp.float32),
        pltpu.VMEM_SHARED((NP, D_HID), jnp.float32),
        pltpu.SemaphoreType.DMA((_NB,)),
        pltpu.SemaphoreType.DMA((_NB,)),
    ])()


_TC_R = 1024


def _tc1_body(x_ref, w_ref, degp_ref, h_ref, u_ref, dinv_ref):
    deg = degp_ref[0] + degp_ref[1] + 1.0
    dinv = lax.rsqrt(jnp.maximum(deg, 1e-12))
    h = jnp.dot(x_ref[...], w_ref[...], preferred_element_type=jnp.float32)
    h_ref[...] = h
    u_ref[...] = h * dinv
    dinv_ref[...] = dinv


def _tc1(x_p, W1, degp):
    grid = NP // _TC_R
    return pl.pallas_call(
        _tc1_body,
        grid=(grid,),
        in_specs=[
            pl.BlockSpec((_TC_R, D_IN), lambda i: (i, 0)),
            pl.BlockSpec((D_IN, D_HID), lambda i: (0, 0)),
            pl.BlockSpec((NC, _TC_R, D_HID), lambda i: (0, i, 0)),
        ],
        out_specs=[
            pl.BlockSpec((_TC_R, D_HID), lambda i: (i, 0)),
            pl.BlockSpec((_TC_R, D_HID), lambda i: (i, 0)),
            pl.BlockSpec((_TC_R, D_HID), lambda i: (i, 0)),
        ],
        out_shape=[
            jax.ShapeDtypeStruct((NP, D_HID), jnp.float32),
            jax.ShapeDtypeStruct((NP, D_HID), jnp.float32),
            jax.ShapeDtypeStruct((NP, D_HID), jnp.float32),
        ],
    )(x_p, W1, degp)


def _tc2_body(s1p_ref, h_ref, dinv_ref, b_ref, z_ref, u2_ref):
    dinv = dinv_ref[...]
    ssum = s1p_ref[0] + s1p_ref[1]
    z = jnp.tanh(dinv * ssum + dinv * dinv * h_ref[...] + b_ref[...])
    z_ref[...] = z
    u2_ref[...] = z * dinv


def _tc2(s1p, h1, dinv, b1r):
    grid = NP // _TC_R
    return pl.pallas_call(
        _tc2_body,
        grid=(grid,),
        in_specs=[
            pl.BlockSpec((NC, _TC_R, D_HID), lambda i: (0, i, 0)),
            pl.BlockSpec((_TC_R, D_HID), lambda i: (i, 0)),
            pl.BlockSpec((_TC_R, D_HID), lambda i: (i, 0)),
            pl.BlockSpec((1, D_HID), lambda i: (0, 0)),
        ],
        out_specs=[
            pl.BlockSpec((_TC_R, D_HID), lambda i: (i, 0)),
            pl.BlockSpec((_TC_R, D_HID), lambda i: (i, 0)),
        ],
        out_shape=[
            jax.ShapeDtypeStruct((NP, D_HID), jnp.float32),
            jax.ShapeDtypeStruct((NP, D_HID), jnp.float32),
        ],
    )(s1p, h1, dinv, b1r)


def _tc3_body(s2p_ref, z_ref, dinv_ref, w_ref, b_ref, o_ref):
    dinv = dinv_ref[...]
    agg = dinv * (s2p_ref[0] + s2p_ref[1]) + dinv * dinv * z_ref[...]
    o_ref[...] = jnp.dot(agg, w_ref[...],
                         preferred_element_type=jnp.float32) + b_ref[...]


def _tc3(s2p, z1, dinv, W2, b2r):
    grid = NP // _TC_R
    return pl.pallas_call(
        _tc3_body,
        grid=(grid,),
        in_specs=[
            pl.BlockSpec((NC, _TC_R, D_HID), lambda i: (0, i, 0)),
            pl.BlockSpec((_TC_R, D_HID), lambda i: (i, 0)),
            pl.BlockSpec((_TC_R, D_HID), lambda i: (i, 0)),
            pl.BlockSpec((D_HID, D_OUT), lambda i: (0, 0)),
            pl.BlockSpec((1, D_OUT), lambda i: (0, 0)),
        ],
        out_specs=pl.BlockSpec((_TC_R, D_OUT), lambda i: (i, 0)),
        out_shape=jax.ShapeDtypeStruct((NP, D_OUT), jnp.float32),
    )(s2p, z1, dinv, W2, b2r)


def kernel(x, edge_index, W1, b1, W2, b2):
    x_p = jnp.pad(x, ((0, NP - N), (0, 0)))
    pad = jnp.full((EP - E,), NP - 1, dtype=jnp.int32)
    src_r = jnp.concatenate([edge_index[0], pad]).reshape(EP // LANES, LANES)
    dst_r = jnp.concatenate([edge_index[1], pad]).reshape(EP // LANES, LANES)
    zeros_tbl = jnp.zeros((NP, D_HID), jnp.float32)
    ones_blk = jnp.ones((LANES, D_HID), jnp.float32)
    b1r = b1.reshape(1, D_HID)
    b2r = b2.reshape(1, D_OUT)

    degp = _deg_call(dst_r, zeros_tbl, ones_blk)
    h1, u1, dinv = _tc1(x_p, W1, degp)
    s1p = _agg_call(u1, src_r, dst_r, zeros_tbl)
    z1, u2 = _tc2(s1p, h1, dinv, b1r)
    s2p = _agg_call(u2, src_r, dst_r, zeros_tbl)
    out_p = _tc3(s2p, z1, dinv, W2, b2r)
    return out_p[:N]


# speedup vs baseline: 1.0907x; 1.0907x over previous
import functools

import jax
import jax.numpy as jnp
from jax import lax
from jax.experimental import pallas as pl
from jax.experimental.pallas import tpu as pltpu
from jax.experimental.pallas import tpu_sc as plsc

N = 10000
D_IN = 128
D_HID = 16
D_OUT = 128
E = 320000

NC = 2
NS = 16
LANES = 128
NP = 10240
EP = 327680
G = EP // (NC * NS * LANES)
G_SC0 = 116
G_SC1 = 2 * G - G_SC0
RPT = NP // NS

_MESH = plsc.VectorSubcoreMesh(
    core_axis_name="c", subcore_axis_name="s", num_cores=NC, num_subcores=NS)


def _stage_idx(ei_hbm, idx_v, c, s):
    @pl.when(c == 0)
    def _():
        pltpu.sync_copy(ei_hbm.at[pl.ds(s * G_SC0, G_SC0)], idx_v)

    @pl.when(c == 1)
    def _():
        pltpu.sync_copy(ei_hbm.at[pl.ds(NS * G_SC0 + s * G_SC1, G_SC1)],
                        idx_v.at[pl.ds(0, G_SC1)])


def _deg_body(ei_hbm, zeros_hbm, ones_hbm, out_hbm, dst_v, ones_v, acc_sh,
              dsem):
    c = lax.axis_index("c")
    s = lax.axis_index("s")
    my_g = lax.select(c == 0, G_SC0, G_SC1)
    _stage_idx(ei_hbm, dst_v, c, s)
    pltpu.sync_copy(zeros_hbm.at[pl.ds(s * RPT, RPT)],
                    acc_sh.at[pl.ds(s * RPT, RPT)])
    pltpu.sync_copy(ones_hbm, ones_v)
    plsc.subcore_barrier()

    def step(g, carry):
        pltpu.async_copy(ones_v, acc_sh.at[dst_v.at[g]], dsem, add=True)
        pltpu.make_async_copy(ones_v, acc_sh.at[dst_v.at[g]], dsem).wait()
        return carry

    lax.fori_loop(0, my_g, step, 0)
    plsc.subcore_barrier()
    pltpu.sync_copy(acc_sh.at[pl.ds(s * RPT, RPT)],
                    out_hbm.at[c, pl.ds(s * RPT, RPT)])


_deg_call = functools.partial(
    pl.kernel, _deg_body, mesh=_MESH,
    compiler_params=pltpu.CompilerParams(use_tc_tiling_on_sc=False),
    out_type=jax.ShapeDtypeStruct((NC, NP, D_HID), jnp.float32),
    scratch_types=[
        pltpu.VMEM((G_SC0, LANES), jnp.int32),
        pltpu.VMEM((LANES, D_HID), jnp.float32),
        pltpu.VMEM_SHARED((NP, D_HID), jnp.float32),
        pltpu.SemaphoreType.DMA,
    ])()


_NB = 4


def _agg_body(u_hbm, src_hbm, dst_hbm, zeros_hbm, out_hbm,
              src_v, dst_v, rows_v, acc_sh, gsem, ssem):
    c = lax.axis_index("c")
    s = lax.axis_index("s")
    my_g = lax.select(c == 0, G_SC0, G_SC1)
    _stage_idx(src_hbm, src_v, c, s)
    _stage_idx(dst_hbm, dst_v, c, s)
    pltpu.sync_copy(zeros_hbm.at[pl.ds(s * RPT, RPT)],
                    acc_sh.at[pl.ds(s * RPT, RPT)])
    plsc.subcore_barrier()

    pltpu.async_copy(u_hbm.at[src_v.at[0]], rows_v.at[0], gsem.at[0])

    def step(g, carry):
        b = lax.rem(g, 2)
        pltpu.make_async_copy(
            u_hbm.at[src_v.at[g]], rows_v.at[b], gsem.at[b]).wait()

        @pl.when(g + 1 < my_g)
        def _():
            pltpu.async_copy(
                u_hbm.at[src_v.at[g + 1]], rows_v.at[1 - b], gsem.at[1 - b])

        pltpu.async_copy(
            rows_v.at[b], acc_sh.at[dst_v.at[g]], ssem.at[0], add=True)
        pltpu.make_async_copy(
            rows_v.at[b], acc_sh.at[dst_v.at[g]], ssem.at[0]).wait()
        return carry

    lax.fori_loop(0, my_g, step, 0)
    plsc.subcore_barrier()
    pltpu.sync_copy(acc_sh.at[pl.ds(s * RPT, RPT)],
                    out_hbm.at[c, pl.ds(s * RPT, RPT)])


_agg_call = functools.partial(
    pl.kernel, _agg_body, mesh=_MESH,
    compiler_params=pltpu.CompilerParams(use_tc_tiling_on_sc=False),
    out_type=jax.ShapeDtypeStruct((NC, NP, D_HID), jnp.float32),
    scratch_types=[
        pltpu.VMEM((G_SC0, LANES), jnp.int32),
        pltpu.VMEM((G_SC0, LANES), jnp.int32),
        pltpu.VMEM((_NB, LANES, D_HID), jnp.float32),
        pltpu.VMEM_SHARED((NP, D_HID), jnp.float32),
        pltpu.SemaphoreType.DMA((_NB,)),
        pltpu.SemaphoreType.DMA((_NB,)),
    ])()


_TC_R = 1024


def _tc1_body(x_ref, w_ref, degp_ref, h_ref, u_ref, dinv_ref):
    deg = degp_ref[0] + degp_ref[1] + 1.0
    dinv = lax.rsqrt(jnp.maximum(deg, 1e-12))
    h = jnp.dot(x_ref[...], w_ref[...], preferred_element_type=jnp.float32)
    h_ref[...] = h
    u_ref[...] = h * dinv
    dinv_ref[...] = dinv


def _tc1(x_p, W1, degp):
    grid = NP // _TC_R
    return pl.pallas_call(
        _tc1_body,
        grid=(grid,),
        in_specs=[
            pl.BlockSpec((_TC_R, D_IN), lambda i: (i, 0)),
            pl.BlockSpec((D_IN, D_HID), lambda i: (0, 0)),
            pl.BlockSpec((NC, _TC_R, D_HID), lambda i: (0, i, 0)),
        ],
        out_specs=[
            pl.BlockSpec((_TC_R, D_HID), lambda i: (i, 0)),
            pl.BlockSpec((_TC_R, D_HID), lambda i: (i, 0)),
            pl.BlockSpec((_TC_R, D_HID), lambda i: (i, 0)),
        ],
        out_shape=[
            jax.ShapeDtypeStruct((NP, D_HID), jnp.float32),
            jax.ShapeDtypeStruct((NP, D_HID), jnp.float32),
            jax.ShapeDtypeStruct((NP, D_HID), jnp.float32),
        ],
    )(x_p, W1, degp)


def _tc2_body(s1p_ref, h_ref, dinv_ref, b_ref, z_ref, u2_ref):
    dinv = dinv_ref[...]
    ssum = s1p_ref[0] + s1p_ref[1]
    z = jnp.tanh(dinv * ssum + dinv * dinv * h_ref[...] + b_ref[...])
    z_ref[...] = z
    u2_ref[...] = z * dinv


def _tc2(s1p, h1, dinv, b1r):
    grid = NP // _TC_R
    return pl.pallas_call(
        _tc2_body,
        grid=(grid,),
        in_specs=[
            pl.BlockSpec((NC, _TC_R, D_HID), lambda i: (0, i, 0)),
            pl.BlockSpec((_TC_R, D_HID), lambda i: (i, 0)),
            pl.BlockSpec((_TC_R, D_HID), lambda i: (i, 0)),
            pl.BlockSpec((1, D_HID), lambda i: (0, 0)),
        ],
        out_specs=[
            pl.BlockSpec((_TC_R, D_HID), lambda i: (i, 0)),
            pl.BlockSpec((_TC_R, D_HID), lambda i: (i, 0)),
        ],
        out_shape=[
            jax.ShapeDtypeStruct((NP, D_HID), jnp.float32),
            jax.ShapeDtypeStruct((NP, D_HID), jnp.float32),
        ],
    )(s1p, h1, dinv, b1r)


def _tc3_body(s2p_ref, z_ref, dinv_ref, w_ref, b_ref, o_ref):
    dinv = dinv_ref[...]
    agg = dinv * (s2p_ref[0] + s2p_ref[1]) + dinv * dinv * z_ref[...]
    o_ref[...] = jnp.dot(agg, w_ref[...],
                         preferred_element_type=jnp.float32) + b_ref[...]


def _tc3(s2p, z1, dinv, W2, b2r):
    grid = NP // _TC_R
    return pl.pallas_call(
        _tc3_body,
        grid=(grid,),
        in_specs=[
            pl.BlockSpec((NC, _TC_R, D_HID), lambda i: (0, i, 0)),
            pl.BlockSpec((_TC_R, D_HID), lambda i: (i, 0)),
            pl.BlockSpec((_TC_R, D_HID), lambda i: (i, 0)),
            pl.BlockSpec((D_HID, D_OUT), lambda i: (0, 0)),
            pl.BlockSpec((1, D_OUT), lambda i: (0, 0)),
        ],
        out_specs=pl.BlockSpec((_TC_R, D_OUT), lambda i: (i, 0)),
        out_shape=jax.ShapeDtypeStruct((NP, D_OUT), jnp.float32),
    )(s2p, z1, dinv, W2, b2r)


def kernel(x, edge_index, W1, b1, W2, b2):
    x_p = jnp.pad(x, ((0, NP - N), (0, 0)))
    pad = jnp.full((EP - E,), NP - 1, dtype=jnp.int32)
    src_r = jnp.concatenate([edge_index[0], pad]).reshape(EP // LANES, LANES)
    dst_r = jnp.concatenate([edge_index[1], pad]).reshape(EP // LANES, LANES)
    zeros_tbl = jnp.zeros((NP, D_HID), jnp.float32)
    ones_blk = jnp.ones((LANES, D_HID), jnp.float32)
    b1r = b1.reshape(1, D_HID)
    b2r = b2.reshape(1, D_OUT)

    degp = _deg_call(dst_r, zeros_tbl, ones_blk)
    h1, u1, dinv = _tc1(x_p, W1, degp)
    s1p = _agg_call(u1, src_r, dst_r, zeros_tbl)
    z1, u2 = _tc2(s1p, h1, dinv, b1r)
    s2p = _agg_call(u2, src_r, dst_r, zeros_tbl)
    out_p = _tc3(s2p, z1, dinv, W2, b2r)
    return out_p[:N]


# speedup vs baseline: 1.2023x; 1.1023x over previous
import functools

import jax
import jax.numpy as jnp
from jax import lax
from jax.experimental import pallas as pl
from jax.experimental.pallas import tpu as pltpu
from jax.experimental.pallas import tpu_sc as plsc

N = 10000
D_IN = 128
D_HID = 16
D_OUT = 128
E = 320000

NC = 2
NS = 16
LANES = 128
NP = 10240
EP = 327680
G = EP // (NC * NS * LANES)
G_SC0 = 94
G_SC1 = 2 * G - G_SC0
RPT = NP // NS

_MESH = plsc.VectorSubcoreMesh(
    core_axis_name="c", subcore_axis_name="s", num_cores=NC, num_subcores=NS)


def _stage_idx(ei_hbm, idx_v, c, s):
    @pl.when(c == 0)
    def _():
        pltpu.sync_copy(ei_hbm.at[pl.ds(s * G_SC0, G_SC0)], idx_v)

    @pl.when(c == 1)
    def _():
        pltpu.sync_copy(ei_hbm.at[pl.ds(NS * G_SC0 + s * G_SC1, G_SC1)],
                        idx_v.at[pl.ds(0, G_SC1)])


def _deg_body(ei_hbm, zeros_hbm, ones_hbm, out_hbm, dst_v, ones_v, acc_sh,
              dsem):
    c = lax.axis_index("c")
    s = lax.axis_index("s")
    my_g = lax.select(c == 0, G_SC0, G_SC1)
    _stage_idx(ei_hbm, dst_v, c, s)
    pltpu.sync_copy(zeros_hbm.at[pl.ds(s * RPT, RPT)],
                    acc_sh.at[pl.ds(s * RPT, RPT)])
    pltpu.sync_copy(ones_hbm, ones_v)
    plsc.subcore_barrier()

    def step(g, carry):
        pltpu.async_copy(ones_v, acc_sh.at[dst_v.at[g]], dsem, add=True)
        pltpu.make_async_copy(ones_v, acc_sh.at[dst_v.at[g]], dsem).wait()
        return carry

    lax.fori_loop(0, my_g, step, 0)
    plsc.subcore_barrier()
    pltpu.sync_copy(acc_sh.at[pl.ds(s * RPT, RPT)],
                    out_hbm.at[c, pl.ds(s * RPT, RPT)])


_deg_call = functools.partial(
    pl.kernel, _deg_body, mesh=_MESH,
    compiler_params=pltpu.CompilerParams(use_tc_tiling_on_sc=False),
    out_type=jax.ShapeDtypeStruct((NC, NP, D_HID), jnp.float32),
    scratch_types=[
        pltpu.VMEM((G_SC0, LANES), jnp.int32),
        pltpu.VMEM((LANES, D_HID), jnp.float32),
        pltpu.VMEM_SHARED((NP, D_HID), jnp.float32),
        pltpu.SemaphoreType.DMA,
    ])()


_NB = 4


def _agg_body(u_hbm, src_hbm, dst_hbm, zeros_hbm, out_hbm,
              src_v, dst_v, rows_v, acc_sh, gsem, ssem):
    c = lax.axis_index("c")
    s = lax.axis_index("s")
    my_g = lax.select(c == 0, G_SC0, G_SC1)
    _stage_idx(src_hbm, src_v, c, s)
    _stage_idx(dst_hbm, dst_v, c, s)
    pltpu.sync_copy(zeros_hbm.at[pl.ds(s * RPT, RPT)],
                    acc_sh.at[pl.ds(s * RPT, RPT)])
    plsc.subcore_barrier()

    pltpu.async_copy(u_hbm.at[src_v.at[0]], rows_v.at[0], gsem.at[0])

    def step(g, carry):
        b = lax.rem(g, 2)
        pltpu.make_async_copy(
            u_hbm.at[src_v.at[g]], rows_v.at[b], gsem.at[b]).wait()
        pltpu.async_copy(
            rows_v.at[b], acc_sh.at[dst_v.at[g]], ssem.at[b], add=True)

        @pl.when(g + 1 < my_g)
        def _():
            @pl.when(g >= 1)
            def _():
                pltpu.make_async_copy(
                    rows_v.at[1 - b], acc_sh.at[dst_v.at[g - 1]],
                    ssem.at[1 - b]).wait()

            pltpu.async_copy(
                u_hbm.at[src_v.at[g + 1]], rows_v.at[1 - b], gsem.at[1 - b])
        return carry

    lax.fori_loop(0, my_g, step, 0)
    bl = lax.rem(my_g - 1, 2)
    pltpu.make_async_copy(
        rows_v.at[1 - bl], acc_sh.at[dst_v.at[my_g - 2]],
        ssem.at[1 - bl]).wait()
    pltpu.make_async_copy(
        rows_v.at[bl], acc_sh.at[dst_v.at[my_g - 1]], ssem.at[bl]).wait()
    plsc.subcore_barrier()
    pltpu.sync_copy(acc_sh.at[pl.ds(s * RPT, RPT)],
                    out_hbm.at[c, pl.ds(s * RPT, RPT)])


_agg_call = functools.partial(
    pl.kernel, _agg_body, mesh=_MESH,
    compiler_params=pltpu.CompilerParams(use_tc_tiling_on_sc=False),
    out_type=jax.ShapeDtypeStruct((NC, NP, D_HID), jnp.float32),
    scratch_types=[
        pltpu.VMEM((G_SC0, LANES), jnp.int32),
        pltpu.VMEM((G_SC0, LANES), jnp.int32),
        pltpu.VMEM((_NB, LANES, D_HID), jnp.float32),
        pltpu.VMEM_SHARED((NP, D_HID), jnp.float32),
        pltpu.SemaphoreType.DMA((_NB,)),
        pltpu.SemaphoreType.DMA((_NB,)),
    ])()


_TC_R = 1024


def _tc1_body(x_ref, w_ref, degp_ref, h_ref, u_ref, dinv_ref):
    deg = degp_ref[0] + degp_ref[1] + 1.0
    dinv = lax.rsqrt(jnp.maximum(deg, 1e-12))
    h = jnp.dot(x_ref[...], w_ref[...], preferred_element_type=jnp.float32)
    h_ref[...] = h
    u_ref[...] = h * dinv
    dinv_ref[...] = dinv


def _tc1(x_p, W1, degp):
    grid = NP // _TC_R
    return pl.pallas_call(
        _tc1_body,
        grid=(grid,),
        in_specs=[
            pl.BlockSpec((_TC_R, D_IN), lambda i: (i, 0)),
            pl.BlockSpec((D_IN, D_HID), lambda i: (0, 0)),
            pl.BlockSpec((NC, _TC_R, D_HID), lambda i: (0, i, 0)),
        ],
        out_specs=[
            pl.BlockSpec((_TC_R, D_HID), lambda i: (i, 0)),
            pl.BlockSpec((_TC_R, D_HID), lambda i: (i, 0)),
            pl.BlockSpec((_TC_R, D_HID), lambda i: (i, 0)),
        ],
        out_shape=[
            jax.ShapeDtypeStruct((NP, D_HID), jnp.float32),
            jax.ShapeDtypeStruct((NP, D_HID), jnp.float32),
            jax.ShapeDtypeStruct((NP, D_HID), jnp.float32),
        ],
    )(x_p, W1, degp)


def _tc2_body(s1p_ref, h_ref, dinv_ref, b_ref, z_ref, u2_ref):
    dinv = dinv_ref[...]
    ssum = s1p_ref[0] + s1p_ref[1]
    z = jnp.tanh(dinv * ssum + dinv * dinv * h_ref[...] + b_ref[...])
    z_ref[...] = z
    u2_ref[...] = z * dinv


def _tc2(s1p, h1, dinv, b1r):
    grid = NP // _TC_R
    return pl.pallas_call(
        _tc2_body,
        grid=(grid,),
        in_specs=[
            pl.BlockSpec((NC, _TC_R, D_HID), lambda i: (0, i, 0)),
            pl.BlockSpec((_TC_R, D_HID), lambda i: (i, 0)),
            pl.BlockSpec((_TC_R, D_HID), lambda i: (i, 0)),
            pl.BlockSpec((1, D_HID), lambda i: (0, 0)),
        ],
        out_specs=[
            pl.BlockSpec((_TC_R, D_HID), lambda i: (i, 0)),
            pl.BlockSpec((_TC_R, D_HID), lambda i: (i, 0)),
        ],
        out_shape=[
            jax.ShapeDtypeStruct((NP, D_HID), jnp.float32),
            jax.ShapeDtypeStruct((NP, D_HID), jnp.float32),
        ],
    )(s1p, h1, dinv, b1r)


def _tc3_body(s2p_ref, z_ref, dinv_ref, w_ref, b_ref, o_ref):
    dinv = dinv_ref[...]
    agg = dinv * (s2p_ref[0] + s2p_ref[1]) + dinv * dinv * z_ref[...]
    o_ref[...] = jnp.dot(agg, w_ref[...],
                         preferred_element_type=jnp.float32) + b_ref[...]


def _tc3(s2p, z1, dinv, W2, b2r):
    grid = NP // _TC_R
    return pl.pallas_call(
        _tc3_body,
        grid=(grid,),
        in_specs=[
            pl.BlockSpec((NC, _TC_R, D_HID), lambda i: (0, i, 0)),
            pl.BlockSpec((_TC_R, D_HID), lambda i: (i, 0)),
            pl.BlockSpec((_TC_R, D_HID), lambda i: (i, 0)),
            pl.BlockSpec((D_HID, D_OUT), lambda i: (0, 0)),
            pl.BlockSpec((1, D_OUT), lambda i: (0, 0)),
        ],
        out_specs=pl.BlockSpec((_TC_R, D_OUT), lambda i: (i, 0)),
        out_shape=jax.ShapeDtypeStruct((NP, D_OUT), jnp.float32),
    )(s2p, z1, dinv, W2, b2r)


def kernel(x, edge_index, W1, b1, W2, b2):
    x_p = jnp.pad(x, ((0, NP - N), (0, 0)))
    pad = jnp.full((EP - E,), NP - 1, dtype=jnp.int32)
    src_r = jnp.concatenate([edge_index[0], pad]).reshape(EP // LANES, LANES)
    dst_r = jnp.concatenate([edge_index[1], pad]).reshape(EP // LANES, LANES)
    zeros_tbl = jnp.zeros((NP, D_HID), jnp.float32)
    ones_blk = jnp.ones((LANES, D_HID), jnp.float32)
    b1r = b1.reshape(1, D_HID)
    b2r = b2.reshape(1, D_OUT)

    degp = _deg_call(dst_r, zeros_tbl, ones_blk)
    h1, u1, dinv = _tc1(x_p, W1, degp)
    s1p = _agg_call(u1, src_r, dst_r, zeros_tbl)
    z1, u2 = _tc2(s1p, h1, dinv, b1r)
    s2p = _agg_call(u2, src_r, dst_r, zeros_tbl)
    out_p = _tc3(s2p, z1, dinv, W2, b2r)
    return out_p[:N]


# speedup vs baseline: 1.4883x; 1.2379x over previous
import functools

import jax
import jax.numpy as jnp
from jax import lax
from jax.experimental import pallas as pl
from jax.experimental.pallas import tpu as pltpu
from jax.experimental.pallas import tpu_sc as plsc

N = 10000
D_IN = 128
D_HID = 16
D_OUT = 128
E = 320000

NC = 2
NS = 16
LANES = 128
NP = 10240
EP = 327680
G = EP // (NC * NS * LANES)
G_SC0 = 94
G_SC1 = 2 * G - G_SC0
RPT = NP // NS

_MESH = plsc.VectorSubcoreMesh(
    core_axis_name="c", subcore_axis_name="s", num_cores=NC, num_subcores=NS)


def _stage_idx(ei_hbm, idx_v, c, s):
    @pl.when(c == 0)
    def _():
        pltpu.sync_copy(ei_hbm.at[pl.ds(s * G_SC0, G_SC0)], idx_v)

    @pl.when(c == 1)
    def _():
        pltpu.sync_copy(ei_hbm.at[pl.ds(NS * G_SC0 + s * G_SC1, G_SC1)],
                        idx_v.at[pl.ds(0, G_SC1)])


def _deg_body(ei_hbm, zeros_hbm, ones_hbm, out_hbm, dst_v, ones_v, acc_sh,
              dsem):
    c = lax.axis_index("c")
    s = lax.axis_index("s")
    my_g = lax.select(c == 0, G_SC0, G_SC1)
    _stage_idx(ei_hbm, dst_v, c, s)
    pltpu.sync_copy(zeros_hbm.at[pl.ds(s * RPT, RPT)],
                    acc_sh.at[pl.ds(s * RPT, RPT)])
    pltpu.sync_copy(ones_hbm, ones_v)
    plsc.subcore_barrier()

    def step(g, carry):
        b = lax.rem(g, 2)

        @pl.when(g >= 2)
        def _():
            pltpu.make_async_copy(
                ones_v, acc_sh.at[dst_v.at[g - 2]], dsem.at[b]).wait()

        pltpu.async_copy(ones_v, acc_sh.at[dst_v.at[g]], dsem.at[b], add=True)
        return carry

    lax.fori_loop(0, my_g, step, 0)
    pltpu.make_async_copy(
        ones_v, acc_sh.at[dst_v.at[my_g - 2]], dsem.at[lax.rem(my_g, 2)]).wait()
    pltpu.make_async_copy(
        ones_v, acc_sh.at[dst_v.at[my_g - 1]],
        dsem.at[lax.rem(my_g - 1, 2)]).wait()
    plsc.subcore_barrier()
    pltpu.sync_copy(acc_sh.at[pl.ds(s * RPT, RPT)],
                    out_hbm.at[c, pl.ds(s * RPT, RPT)])


_deg_call = functools.partial(
    pl.kernel, _deg_body, mesh=_MESH,
    compiler_params=pltpu.CompilerParams(use_tc_tiling_on_sc=False),
    out_type=jax.ShapeDtypeStruct((NC, NP, D_HID), jnp.float32),
    scratch_types=[
        pltpu.VMEM((G_SC0, LANES), jnp.int32),
        pltpu.VMEM((LANES, D_HID), jnp.float32),
        pltpu.VMEM_SHARED((NP, D_HID), jnp.float32),
        pltpu.SemaphoreType.DMA((2,)),
    ])()


_NB = 4


def _agg_body(u_hbm, src_hbm, dst_hbm, zeros_hbm, out_hbm,
              src_v, dst_v, rows_v, acc_sh, gsem, ssem):
    c = lax.axis_index("c")
    s = lax.axis_index("s")
    my_g = lax.select(c == 0, G_SC0, G_SC1)
    _stage_idx(src_hbm, src_v, c, s)
    _stage_idx(dst_hbm, dst_v, c, s)
    pltpu.sync_copy(zeros_hbm.at[pl.ds(s * RPT, RPT)],
                    acc_sh.at[pl.ds(s * RPT, RPT)])
    plsc.subcore_barrier()

    pltpu.async_copy(u_hbm.at[src_v.at[0]], rows_v.at[0], gsem.at[0])
    pltpu.async_copy(u_hbm.at[src_v.at[1]], rows_v.at[1], gsem.at[1])

    def step(g, carry):
        b = lax.rem(g, 3)
        pltpu.make_async_copy(
            u_hbm.at[src_v.at[g]], rows_v.at[b], gsem.at[b]).wait()
        pltpu.async_copy(
            rows_v.at[b], acc_sh.at[dst_v.at[g]], ssem.at[b], add=True)

        @pl.when(g + 2 < my_g)
        def _():
            b2 = lax.rem(g + 2, 3)

            @pl.when(g >= 1)
            def _():
                pltpu.make_async_copy(
                    rows_v.at[b2], acc_sh.at[dst_v.at[g - 1]],
                    ssem.at[b2]).wait()

            pltpu.async_copy(
                u_hbm.at[src_v.at[g + 2]], rows_v.at[b2], gsem.at[b2])
        return carry

    lax.fori_loop(0, my_g, step, 0)
    for k in range(3):
        g_last = my_g - 3 + k
        pltpu.make_async_copy(
            rows_v.at[lax.rem(g_last, 3)], acc_sh.at[dst_v.at[g_last]],
            ssem.at[lax.rem(g_last, 3)]).wait()
    plsc.subcore_barrier()
    pltpu.sync_copy(acc_sh.at[pl.ds(s * RPT, RPT)],
                    out_hbm.at[c, pl.ds(s * RPT, RPT)])


_agg_call = functools.partial(
    pl.kernel, _agg_body, mesh=_MESH,
    compiler_params=pltpu.CompilerParams(use_tc_tiling_on_sc=False),
    out_type=jax.ShapeDtypeStruct((NC, NP, D_HID), jnp.float32),
    scratch_types=[
        pltpu.VMEM((G_SC0, LANES), jnp.int32),
        pltpu.VMEM((G_SC0, LANES), jnp.int32),
        pltpu.VMEM((_NB, LANES, D_HID), jnp.float32),
        pltpu.VMEM_SHARED((NP, D_HID), jnp.float32),
        pltpu.SemaphoreType.DMA((_NB,)),
        pltpu.SemaphoreType.DMA((_NB,)),
    ])()


_TC_R = 1024


def _tc1_body(x_ref, w_ref, degp_ref, h_ref, u_ref, dinv_ref):
    deg = degp_ref[0] + degp_ref[1] + 1.0
    dinv = lax.rsqrt(jnp.maximum(deg, 1e-12))
    h = jnp.dot(x_ref[...], w_ref[...], preferred_element_type=jnp.float32)
    h_ref[...] = h
    u_ref[...] = h * dinv
    dinv_ref[...] = dinv


def _tc1(x_p, W1, degp):
    grid = NP // _TC_R
    return pl.pallas_call(
        _tc1_body,
        grid=(grid,),
        in_specs=[
            pl.BlockSpec((_TC_R, D_IN), lambda i: (i, 0)),
            pl.BlockSpec((D_IN, D_HID), lambda i: (0, 0)),
            pl.BlockSpec((NC, _TC_R, D_HID), lambda i: (0, i, 0)),
        ],
        out_specs=[
            pl.BlockSpec((_TC_R, D_HID), lambda i: (i, 0)),
            pl.BlockSpec((_TC_R, D_HID), lambda i: (i, 0)),
            pl.BlockSpec((_TC_R, D_HID), lambda i: (i, 0)),
        ],
        out_shape=[
            jax.ShapeDtypeStruct((NP, D_HID), jnp.float32),
            jax.ShapeDtypeStruct((NP, D_HID), jnp.float32),
            jax.ShapeDtypeStruct((NP, D_HID), jnp.float32),
        ],
    )(x_p, W1, degp)


def _tc2_body(s1p_ref, h_ref, dinv_ref, b_ref, z_ref, u2_ref):
    dinv = dinv_ref[...]
    ssum = s1p_ref[0] + s1p_ref[1]
    z = jnp.tanh(dinv * ssum + dinv * dinv * h_ref[...] + b_ref[...])
    z_ref[...] = z
    u2_ref[...] = z * dinv


def _tc2(s1p, h1, dinv, b1r):
    grid = NP // _TC_R
    return pl.pallas_call(
        _tc2_body,
        grid=(grid,),
        in_specs=[
            pl.BlockSpec((NC, _TC_R, D_HID), lambda i: (0, i, 0)),
            pl.BlockSpec((_TC_R, D_HID), lambda i: (i, 0)),
            pl.BlockSpec((_TC_R, D_HID), lambda i: (i, 0)),
            pl.BlockSpec((1, D_HID), lambda i: (0, 0)),
        ],
        out_specs=[
            pl.BlockSpec((_TC_R, D_HID), lambda i: (i, 0)),
            pl.BlockSpec((_TC_R, D_HID), lambda i: (i, 0)),
        ],
        out_shape=[
            jax.ShapeDtypeStruct((NP, D_HID), jnp.float32),
            jax.ShapeDtypeStruct((NP, D_HID), jnp.float32),
        ],
    )(s1p, h1, dinv, b1r)


def _tc3_body(s2p_ref, z_ref, dinv_ref, w_ref, b_ref, o_ref):
    dinv = dinv_ref[...]
    agg = dinv * (s2p_ref[0] + s2p_ref[1]) + dinv * dinv * z_ref[...]
    o_ref[...] = jnp.dot(agg, w_ref[...],
                         preferred_element_type=jnp.float32) + b_ref[...]


def _tc3(s2p, z1, dinv, W2, b2r):
    grid = NP // _TC_R
    return pl.pallas_call(
        _tc3_body,
        grid=(grid,),
        in_specs=[
            pl.BlockSpec((NC, _TC_R, D_HID), lambda i: (0, i, 0)),
            pl.BlockSpec((_TC_R, D_HID), lambda i: (i, 0)),
            pl.BlockSpec((_TC_R, D_HID), lambda i: (i, 0)),
            pl.BlockSpec((D_HID, D_OUT), lambda i: (0, 0)),
            pl.BlockSpec((1, D_OUT), lambda i: (0, 0)),
        ],
        out_specs=pl.BlockSpec((_TC_R, D_OUT), lambda i: (i, 0)),
        out_shape=jax.ShapeDtypeStruct((NP, D_OUT), jnp.float32),
    )(s2p, z1, dinv, W2, b2r)


def kernel(x, edge_index, W1, b1, W2, b2):
    x_p = jnp.pad(x, ((0, NP - N), (0, 0)))
    pad = jnp.full((EP - E,), NP - 1, dtype=jnp.int32)
    src_r = jnp.concatenate([edge_index[0], pad]).reshape(EP // LANES, LANES)
    dst_r = jnp.concatenate([edge_index[1], pad]).reshape(EP // LANES, LANES)
    zeros_tbl = jnp.zeros((NP, D_HID), jnp.float32)
    ones_blk = jnp.ones((LANES, D_HID), jnp.float32)
    b1r = b1.reshape(1, D_HID)
    b2r = b2.reshape(1, D_OUT)

    degp = _deg_call(dst_r, zeros_tbl, ones_blk)
    h1, u1, dinv = _tc1(x_p, W1, degp)
    s1p = _agg_call(u1, src_r, dst_r, zeros_tbl)
    z1, u2 = _tc2(s1p, h1, dinv, b1r)
    s2p = _agg_call(u2, src_r, dst_r, zeros_tbl)
    out_p = _tc3(s2p, z1, dinv, W2, b2r)
    return out_p[:N]


# speedup vs baseline: 1.4933x; 1.0033x over previous
import functools

import jax
import jax.numpy as jnp
from jax import lax
from jax.experimental import pallas as pl
from jax.experimental.pallas import tpu as pltpu
from jax.experimental.pallas import tpu_sc as plsc

N = 10000
D_IN = 128
D_HID = 16
D_OUT = 128
E = 320000

NC = 2
NS = 16
LANES = 128
NP = 10240
EP = 327680
G = EP // (NC * NS * LANES)
G_SC0 = 94
G_SC1 = 2 * G - G_SC0
RPT = NP // NS

_MESH = plsc.VectorSubcoreMesh(
    core_axis_name="c", subcore_axis_name="s", num_cores=NC, num_subcores=NS)


def _stage_idx(ei_hbm, idx_v, c, s):
    @pl.when(c == 0)
    def _():
        pltpu.sync_copy(ei_hbm.at[pl.ds(s * G_SC0, G_SC0)], idx_v)

    @pl.when(c == 1)
    def _():
        pltpu.sync_copy(ei_hbm.at[pl.ds(NS * G_SC0 + s * G_SC1, G_SC1)],
                        idx_v.at[pl.ds(0, G_SC1)])


def _deg_body(ei_hbm, zeros_hbm, ones_hbm, out_hbm, dst_v, ones_v, acc_sh,
              dsem):
    c = lax.axis_index("c")
    s = lax.axis_index("s")
    my_g = lax.select(c == 0, G_SC0, G_SC1)
    _stage_idx(ei_hbm, dst_v, c, s)
    pltpu.sync_copy(zeros_hbm.at[pl.ds(s * RPT, RPT)],
                    acc_sh.at[pl.ds(s * RPT, RPT)])
    pltpu.sync_copy(ones_hbm, ones_v)
    plsc.subcore_barrier()

    def step(g, carry):
        b = lax.rem(g, 2)

        @pl.when(g >= 2)
        def _():
            pltpu.make_async_copy(
                ones_v, acc_sh.at[dst_v.at[g - 2]], dsem.at[b]).wait()

        pltpu.async_copy(ones_v, acc_sh.at[dst_v.at[g]], dsem.at[b], add=True)
        return carry

    lax.fori_loop(0, my_g, step, 0)
    pltpu.make_async_copy(
        ones_v, acc_sh.at[dst_v.at[my_g - 2]], dsem.at[lax.rem(my_g, 2)]).wait()
    pltpu.make_async_copy(
        ones_v, acc_sh.at[dst_v.at[my_g - 1]],
        dsem.at[lax.rem(my_g - 1, 2)]).wait()
    plsc.subcore_barrier()
    pltpu.sync_copy(acc_sh.at[pl.ds(s * RPT, RPT)],
                    out_hbm.at[c, pl.ds(s * RPT, RPT)])


_deg_call = functools.partial(
    pl.kernel, _deg_body, mesh=_MESH,
    compiler_params=pltpu.CompilerParams(use_tc_tiling_on_sc=False),
    out_type=jax.ShapeDtypeStruct((NC, NP, D_HID), jnp.float32),
    scratch_types=[
        pltpu.VMEM((G_SC0, LANES), jnp.int32),
        pltpu.VMEM((LANES, D_HID), jnp.float32),
        pltpu.VMEM_SHARED((NP, D_HID), jnp.float32),
        pltpu.SemaphoreType.DMA((2,)),
    ])()


_NB = 4


def _agg_body(u_hbm, src_hbm, dst_hbm, zeros_hbm, out_hbm,
              src_v, dst_v, rows_v, acc_sh, gsem, ssem):
    c = lax.axis_index("c")
    s = lax.axis_index("s")
    my_g = lax.select(c == 0, G_SC0, G_SC1)
    _stage_idx(src_hbm, src_v, c, s)
    _stage_idx(dst_hbm, dst_v, c, s)
    pltpu.sync_copy(zeros_hbm.at[pl.ds(s * RPT, RPT)],
                    acc_sh.at[pl.ds(s * RPT, RPT)])
    plsc.subcore_barrier()

    for b in range(_NB - 1):
        pltpu.async_copy(u_hbm.at[src_v.at[b]], rows_v.at[b], gsem.at[b])

    def step(g, carry):
        b = lax.rem(g, _NB)
        pltpu.make_async_copy(
            u_hbm.at[src_v.at[g]], rows_v.at[b], gsem.at[b]).wait()
        pltpu.async_copy(
            rows_v.at[b], acc_sh.at[dst_v.at[g]], ssem.at[b], add=True)

        @pl.when(g + _NB - 1 < my_g)
        def _():
            b2 = lax.rem(g + _NB - 1, _NB)

            @pl.when(g >= 1)
            def _():
                pltpu.make_async_copy(
                    rows_v.at[b2], acc_sh.at[dst_v.at[g - 1]],
                    ssem.at[b2]).wait()

            pltpu.async_copy(
                u_hbm.at[src_v.at[g + _NB - 1]], rows_v.at[b2], gsem.at[b2])
        return carry

    lax.fori_loop(0, my_g, step, 0)
    for k in range(_NB):
        g_last = my_g - _NB + k
        pltpu.make_async_copy(
            rows_v.at[lax.rem(g_last, _NB)], acc_sh.at[dst_v.at[g_last]],
            ssem.at[lax.rem(g_last, _NB)]).wait()
    plsc.subcore_barrier()
    pltpu.sync_copy(acc_sh.at[pl.ds(s * RPT, RPT)],
                    out_hbm.at[c, pl.ds(s * RPT, RPT)])


_agg_call = functools.partial(
    pl.kernel, _agg_body, mesh=_MESH,
    compiler_params=pltpu.CompilerParams(use_tc_tiling_on_sc=False),
    out_type=jax.ShapeDtypeStruct((NC, NP, D_HID), jnp.float32),
    scratch_types=[
        pltpu.VMEM((G_SC0, LANES), jnp.int32),
        pltpu.VMEM((G_SC0, LANES), jnp.int32),
        pltpu.VMEM((_NB, LANES, D_HID), jnp.float32),
        pltpu.VMEM_SHARED((NP, D_HID), jnp.float32),
        pltpu.SemaphoreType.DMA((_NB,)),
        pltpu.SemaphoreType.DMA((_NB,)),
    ])()


_TC_R = 1024


def _tc1_body(x_ref, w_ref, degp_ref, h_ref, u_ref, dinv_ref):
    deg = degp_ref[0] + degp_ref[1] + 1.0
    dinv = lax.rsqrt(jnp.maximum(deg, 1e-12))
    h = jnp.dot(x_ref[...], w_ref[...], preferred_element_type=jnp.float32)
    h_ref[...] = h
    u_ref[...] = h * dinv
    dinv_ref[...] = dinv


def _tc1(x_p, W1, degp):
    grid = NP // _TC_R
    return pl.pallas_call(
        _tc1_body,
        grid=(grid,),
        in_specs=[
            pl.BlockSpec((_TC_R, D_IN), lambda i: (i, 0)),
            pl.BlockSpec((D_IN, D_HID), lambda i: (0, 0)),
            pl.BlockSpec((NC, _TC_R, D_HID), lambda i: (0, i, 0)),
        ],
        out_specs=[
            pl.BlockSpec((_TC_R, D_HID), lambda i: (i, 0)),
            pl.BlockSpec((_TC_R, D_HID), lambda i: (i, 0)),
            pl.BlockSpec((_TC_R, D_HID), lambda i: (i, 0)),
        ],
        out_shape=[
            jax.ShapeDtypeStruct((NP, D_HID), jnp.float32),
            jax.ShapeDtypeStruct((NP, D_HID), jnp.float32),
            jax.ShapeDtypeStruct((NP, D_HID), jnp.float32),
        ],
    )(x_p, W1, degp)


def _tc2_body(s1p_ref, h_ref, dinv_ref, b_ref, z_ref, u2_ref):
    dinv = dinv_ref[...]
    ssum = s1p_ref[0] + s1p_ref[1]
    z = jnp.tanh(dinv * ssum + dinv * dinv * h_ref[...] + b_ref[...])
    z_ref[...] = z
    u2_ref[...] = z * dinv


def _tc2(s1p, h1, dinv, b1r):
    grid = NP // _TC_R
    return pl.pallas_call(
        _tc2_body,
        grid=(grid,),
        in_specs=[
            pl.BlockSpec((NC, _TC_R, D_HID), lambda i: (0, i, 0)),
            pl.BlockSpec((_TC_R, D_HID), lambda i: (i, 0)),
            pl.BlockSpec((_TC_R, D_HID), lambda i: (i, 0)),
            pl.BlockSpec((1, D_HID), lambda i: (0, 0)),
        ],
        out_specs=[
            pl.BlockSpec((_TC_R, D_HID), lambda i: (i, 0)),
            pl.BlockSpec((_TC_R, D_HID), lambda i: (i, 0)),
        ],
        out_shape=[
            jax.ShapeDtypeStruct((NP, D_HID), jnp.float32),
            jax.ShapeDtypeStruct((NP, D_HID), jnp.float32),
        ],
    )(s1p, h1, dinv, b1r)


def _tc3_body(s2p_ref, z_ref, dinv_ref, w_ref, b_ref, o_ref):
    dinv = dinv_ref[...]
    agg = dinv * (s2p_ref[0] + s2p_ref[1]) + dinv * dinv * z_ref[...]
    o_ref[...] = jnp.dot(agg, w_ref[...],
                         preferred_element_type=jnp.float32) + b_ref[...]


def _tc3(s2p, z1, dinv, W2, b2r):
    grid = NP // _TC_R
    return pl.pallas_call(
        _tc3_body,
        grid=(grid,),
        in_specs=[
            pl.BlockSpec((NC, _TC_R, D_HID), lambda i: (0, i, 0)),
            pl.BlockSpec((_TC_R, D_HID), lambda i: (i, 0)),
            pl.BlockSpec((_TC_R, D_HID), lambda i: (i, 0)),
            pl.BlockSpec((D_HID, D_OUT), lambda i: (0, 0)),
            pl.BlockSpec((1, D_OUT), lambda i: (0, 0)),
        ],
        out_specs=pl.BlockSpec((_TC_R, D_OUT), lambda i: (i, 0)),
        out_shape=jax.ShapeDtypeStruct((NP, D_OUT), jnp.float32),
    )(s2p, z1, dinv, W2, b2r)


def kernel(x, edge_index, W1, b1, W2, b2):
    x_p = jnp.pad(x, ((0, NP - N), (0, 0)))
    pad = jnp.full((EP - E,), NP - 1, dtype=jnp.int32)
    src_r = jnp.concatenate([edge_index[0], pad]).reshape(EP // LANES, LANES)
    dst_r = jnp.concatenate([edge_index[1], pad]).reshape(EP // LANES, LANES)
    zeros_tbl = jnp.zeros((NP, D_HID), jnp.float32)
    ones_blk = jnp.ones((LANES, D_HID), jnp.float32)
    b1r = b1.reshape(1, D_HID)
    b2r = b2.reshape(1, D_OUT)

    degp = _deg_call(dst_r, zeros_tbl, ones_blk)
    h1, u1, dinv = _tc1(x_p, W1, degp)
    s1p = _agg_call(u1, src_r, dst_r, zeros_tbl)
    z1, u2 = _tc2(s1p, h1, dinv, b1r)
    s2p = _agg_call(u2, src_r, dst_r, zeros_tbl)
    out_p = _tc3(s2p, z1, dinv, W2, b2r)
    return out_p[:N]
